# baseline (speedup 1.0000x reference)
import functools

import jax
import jax.numpy as jnp
from jax import lax
from jax.experimental import pallas as pl
from jax.experimental.pallas import tpu as pltpu
from jax.experimental.pallas import tpu_sc as plsc

N = 10000
E = 320000
D = 128
H = 128
G = 128

_NSC = 2
_NTILE = 16
_KCH = 128
NP = N + 112


def _mm_body(x_ref, w_ref, b_ref, o_ref, *, relu):
    acc = jnp.dot(x_ref[...], w_ref[...], preferred_element_type=jnp.float32)
    acc = acc + b_ref[...]
    o_ref[...] = jnp.maximum(acc, 0.0) if relu else acc


def mm_bias(x, w, b, relu=False):
    m = x.shape[0]
    blk = 1000 if m % 1000 == 0 else m
    return pl.pallas_call(
        functools.partial(_mm_body, relu=relu),
        grid=(m // blk,),
        in_specs=[
            pl.BlockSpec((blk, H), lambda i: (i, 0)),
            pl.BlockSpec((H, H), lambda i: (0, 0)),
            pl.BlockSpec((1, H), lambda i: (0, 0)),
        ],
        out_specs=pl.BlockSpec((blk, H), lambda i: (i, 0)),
        out_shape=jax.ShapeDtypeStruct((m, H), jnp.float32),
    )(x, w, b.reshape(1, H))


def _make_prop(n_acc, e_pad):
    ept = e_pad // (_NSC * _NTILE)
    iters = ept // _KCH
    rpt = n_acc // _NTILE
    mesh = plsc.VectorSubcoreMesh(core_axis_name="c", subcore_axis_name="s")

    @functools.partial(
        pl.kernel,
        mesh=mesh,
        out_type=jax.ShapeDtypeStruct((_NSC, n_acc, H), jnp.float32),
        scratch_types=[
            pltpu.VMEM((_KCH,), jnp.int32),
            pltpu.VMEM((_KCH,), jnp.int32),
            pltpu.VMEM((_KCH, H), jnp.float32),
            pltpu.VMEM_SHARED((n_acc, H), jnp.float32),
            pltpu.SemaphoreType.DMA,
        ],
    )
    def prop(feat, srcp, dstp, zrows, out, src_v, dst_v, rows_v, acc, sem):
        c = lax.axis_index("c")
        s = lax.axis_index("s")
        pltpu.sync_copy(zrows, acc.at[pl.ds(s * rpt, rpt)])
        plsc.subcore_barrier()
        base = (c * _NTILE + s) * ept

        def body(i, carry):
            off = base + i * _KCH
            pltpu.sync_copy(srcp.at[pl.ds(off, _KCH)], src_v)
            pltpu.sync_copy(dstp.at[pl.ds(off, _KCH)], dst_v)
            pltpu.async_copy(feat.at[src_v], rows_v, sem).wait()
            pltpu.sync_copy(rows_v, acc.at[dst_v], add=True)
            return carry

        lax.fori_loop(0, iters, body, 0)
        plsc.subcore_barrier()
        pltpu.sync_copy(acc.at[pl.ds(s * rpt, rpt)],
                        out.at[c, pl.ds(s * rpt, rpt)])

    return prop


_PROP_CACHE = {}


def _prop_sc(feat_pad, srcp, dstp, n_acc):
    e_pad = srcp.shape[0]
    key = (n_acc, e_pad)
    if key not in _PROP_CACHE:
        _PROP_CACHE[key] = _make_prop(n_acc, e_pad)
    rpt = n_acc // _NTILE
    zrows = jnp.zeros((rpt, H), jnp.float32)
    parts = _PROP_CACHE[key](feat_pad, srcp, dstp, zrows)
    return parts[0] + parts[1]


def _pad_edges(src, dst, trash):
    e = src.shape[0]
    e_pad = -(-e // (_NSC * _NTILE * _KCH)) * (_NSC * _NTILE * _KCH)
    pad = e_pad - e
    srcp = jnp.concatenate([src, jnp.full((pad,), trash, jnp.int32)])
    dstp = jnp.concatenate([dst, jnp.full((pad,), trash, jnp.int32)])
    return srcp, dstp


def propagate(feat, src, dst, n_out):
    return jnp.zeros((n_out, feat.shape[1]), feat.dtype).at[dst].add(feat[src])


def segsum(z, batch):
    return jnp.zeros((G, z.shape[1]), z.dtype).at[batch].add(z)


def _degree(dst, ew, n):
    deg = jnp.zeros((n,), jnp.float32).at[dst].add(ew)
    return jnp.maximum(deg, 1.0)


def _scale_pad(u, dinv):
    return jnp.zeros((NP, H), jnp.float32).at[:N].set(u * dinv[:, None])


def _encode_pair(u1, dinv, srcp, dstp, b1, W2, b2):
    v1 = _scale_pad(u1, dinv)
    agg1 = _prop_sc(v1, srcp, dstp, NP)[:N] * dinv[:, None]
    h = jnp.maximum(agg1 + b1[None, :], 0.0)
    u2 = mm_bias(h, W2, jnp.zeros((H,), jnp.float32))
    v2 = _scale_pad(u2, dinv)
    z = _prop_sc(v2, srcp, dstp, NP)[:N] * dinv[:, None] + b2[None, :]
    return z


def kernel(x, edge_index, batch, W1, b1, W2, b2):
    src = edge_index[0]
    dst = edge_index[1]

    akey = jax.random.key(42)
    ka, kb, kw = jax.random.split(akey, 3)
    fmask = jax.random.bernoulli(ka, 0.8, (1, D)).astype(x.dtype)
    x1 = x * fmask
    ew2 = jax.random.bernoulli(kb, 0.8, (E,)).astype(x.dtype)

    u_a = mm_bias(x, W1, jnp.zeros((H,), jnp.float32))
    w1m = W1 * fmask[0][:, None]
    u_b = mm_bias(x, w1m, jnp.zeros((H,), jnp.float32))

    deg1 = _degree(dst, jnp.ones((E,), jnp.float32), N)
    dinv1 = lax.rsqrt(deg1)
    deg2 = _degree(dst, ew2, N)
    dinv2 = lax.rsqrt(deg2)

    srcp1, dstp1 = _pad_edges(src, dst, N)
    keep = ew2 > 0.5
    srcp2, dstp2 = _pad_edges(jnp.where(keep, src, N), jnp.where(keep, dst, N), N)

    z = _encode_pair(u_a, dinv1, srcp1, dstp1, b1, W2, b2)
    g = segsum(z, batch)
    z1 = _encode_pair(u_b, dinv1, srcp1, dstp1, b1, W2, b2)
    g1 = segsum(z1, batch)
    z2 = _encode_pair(u_a, dinv2, srcp2, dstp2, b1, W2, b2)
    g2 = segsum(z2, batch)

    order = jnp.argsort(src)
    src_s = src[order]
    dst_s = dst[order]
    rowptr = jnp.searchsorted(src_s, jnp.arange(N + 1, dtype=jnp.int32)).astype(jnp.int32)

    def walk(key, batch_size, length):
        k0 = jax.random.fold_in(key, 10000)
        cur = jax.random.randint(k0, (batch_size,), 0, N, dtype=jnp.int32)
        es, ed = [], []
        for i in range(length):
            ki = jax.random.fold_in(key, i)
            left = rowptr[cur]
            degc = rowptr[cur + 1] - left
            r = jax.random.randint(ki, (batch_size,), 0, 1 << 30, dtype=jnp.int32)
            idx = jnp.clip(left + r % jnp.maximum(degc, 1), 0, E - 1)
            nxt = jnp.where(degc > 0, dst_s[idx], cur)
            es.append(cur)
            ed.append(nxt)
            cur = nxt
        return jnp.concatenate(es), jnp.concatenate(ed)

    def rw_encode(s, d):
        degw = _degree(d, jnp.ones((s.shape[0],), jnp.float32), N)
        dinvw = lax.rsqrt(degw)
        sp, dp = _pad_edges(s, d, N)
        zw = _encode_pair(u_a, dinvw, sp, dp, b1, W2, b2)
        return segsum(zw, batch)

    gs3, gs4 = [], []
    for num in range(3):
        k3 = jax.random.fold_in(kw, 2 * num)
        k4 = jax.random.fold_in(kw, 2 * num + 1)
        s3, d3 = walk(k3, 1000, 7 + num)
        s4, d4 = walk(k4, 999, 12 + num)
        gs3.append(rw_encode(s3, d3))
        gs4.append(rw_encode(s4, d4))

    return (z, g, z1, z2, g1, g2, x1, x, tuple(gs3), tuple(gs4))
